# baseline (speedup 1.0000x reference)
import functools

import jax
import jax.numpy as jnp
from jax import lax
from jax.experimental import pallas as pl
from jax.experimental.pallas import tpu as pltpu
from jax.experimental.pallas import tpu_sc as plsc

_N = 16384
_C = 1000
_CB = 1024
_NC = 2
_NS = 16
_NW = _NC * _NS
_RW = _N // _NW
_CH = 32
_NCHUNK = _RW // _CH
_L = 16
_IR = 128


def _sc_body(x_hbm, lab_hbm, sums_hbm, cnt_hbm,
             xbuf, lab_v, pe_v, ones_v, zero_v, sh_sums, sh_cnt, dma_sems):
    cid = lax.axis_index("c")
    sid = lax.axis_index("s")
    wid = sid * _NC + cid
    base = wid * _RW

    for j in range(_RW // _IR):
        pltpu.sync_copy(lab_hbm.at[pl.ds(base + j * _IR, _IR)], lab_v.at[j])

    for j in range(_RW // _IR):
        for i in range(_IR // _L):
            ones_v[j, pl.ds(i * _L, _L)] = jnp.ones((_L,), jnp.float32)

    @pl.when(sid == 0)
    def _():
        for i in range(_CB // _L):
            zero_v[pl.ds(i * _L, _L)] = jnp.zeros((_L,), jnp.float32)
        pltpu.sync_copy(zero_v, sh_sums)
        pltpu.sync_copy(zero_v, sh_cnt)

    lane = lax.broadcasted_iota(jnp.int32, (_L,), 0)
    zero16 = jnp.zeros((_L,), jnp.float32)

    cps = [None] * _NCHUNK
    cps[0] = pltpu.async_copy(x_hbm.at[pl.ds(base, _CH)], xbuf.at[0],
                              dma_sems.at[0])
    for k in range(_NCHUNK):
        if k + 1 < _NCHUNK:
            cps[k + 1] = pltpu.async_copy(
                x_hbm.at[pl.ds(base + (k + 1) * _CH, _CH)],
                xbuf.at[(k + 1) % 2], dma_sems.at[(k + 1) % 2])
        cps[k].wait()
        buf = k % 2
        for g in range(_CH // _L):
            rows = g * _L + lane

            def col_step(j, carry, _buf=buf, _rows=rows):
                a0, a1, a2, a3, c0, c1, c2, c3 = carry
                v0 = plsc.load_gather(xbuf.at[_buf], [_rows, c0])
                v1 = plsc.load_gather(xbuf.at[_buf], [_rows, c1])
                v2 = plsc.load_gather(xbuf.at[_buf], [_rows, c2])
                v3 = plsc.load_gather(xbuf.at[_buf], [_rows, c3])
                return (a0 + v0 * v0, a1 + v1 * v1,
                        a2 + v2 * v2, a3 + v3 * v3,
                        c0 + 4, c1 + 4, c2 + 4, c3 + 4)

            zl = lane * 0
            a0, a1, a2, a3, _, _, _, _ = lax.fori_loop(
                0, _C // 4, col_step,
                (zero16, zero16, zero16, zero16, zl, zl + 1, zl + 2, zl + 3))
            ssq = (a0 + a1) + (a2 + a3)
            o = k * _CH + g * _L
            lvec = lab_v[o // _IR, pl.ds(o % _IR, _L)]
            gval = plsc.load_gather(xbuf.at[buf], [rows, lvec])
            pe = (ssq - 2.0 * gval + 1.0) * (1.0 / _C)
            pe_v[o // _IR, pl.ds(o % _IR, _L)] = pe

    plsc.subcore_barrier()
    for j in range(_RW // _IR):
        pltpu.sync_copy(pe_v.at[j], sh_sums.at[lab_v.at[j]], add=True)
        pltpu.sync_copy(ones_v.at[j], sh_cnt.at[lab_v.at[j]], add=True)
    plsc.subcore_barrier()

    @pl.when(sid == 0)
    def _():
        pltpu.sync_copy(sh_sums, sums_hbm.at[cid])
        pltpu.sync_copy(sh_cnt, cnt_hbm.at[cid])


@functools.partial(pl.kernel,
                   out_type=[jax.ShapeDtypeStruct((_NC, _CB), jnp.float32),
                             jax.ShapeDtypeStruct((_NC, _CB), jnp.float32)],
                   mesh=plsc.VectorSubcoreMesh(core_axis_name="c",
                                               subcore_axis_name="s"),
                   compiler_params=pltpu.CompilerParams(
                       use_tc_tiling_on_sc=False,
                       needs_layout_passes=False),
                   scratch_types=[
                       pltpu.VMEM((2, _CH, _C), jnp.float32),
                       pltpu.VMEM((_RW // _IR, _IR), jnp.int32),
                       pltpu.VMEM((_RW // _IR, _IR), jnp.float32),
                       pltpu.VMEM((_RW // _IR, _IR), jnp.float32),
                       pltpu.VMEM((_CB,), jnp.float32),
                       pltpu.VMEM_SHARED((_CB,), jnp.float32),
                       pltpu.VMEM_SHARED((_CB,), jnp.float32),
                       pltpu.SemaphoreType.DMA((2,)),
                   ])
def _sc_kernel(x_hbm, lab_hbm, sums_hbm, cnt_hbm, *rest):
    _sc_body(x_hbm, lab_hbm, sums_hbm, cnt_hbm, *rest)


@jax.jit
def kernel(inputs, labels):
    sums2, cnt2 = _sc_kernel(inputs, labels.astype(jnp.int32))
    sums = (sums2[0] + sums2[1])[:_C]
    cnt = (cnt2[0] + cnt2[1])[:_C]
    return (sums, cnt)


# speedup vs baseline: 2.0775x; 2.0775x over previous
import functools

import jax
import jax.numpy as jnp
from jax.experimental import pallas as pl

_N = 16384
_C = 1000
_B = 1024
_G = _N // (2 * _B)


def _body(lab_ref, lab2_ref, x_ref, x2_ref, sums_ref, cnt_ref):
    acc_a = None
    for x, lab in ((x_ref[...], lab_ref[...]), (x2_ref[...], lab2_ref[...])):
        col = jax.lax.broadcasted_iota(jnp.int32, (_B, _C), 1)
        onehot = col == lab
        sumsq1 = jnp.sum(x * x, axis=1, keepdims=True) + 1.0
        a = jnp.sum(jnp.where(onehot, sumsq1 - 2.0 * x, 0.0), axis=0,
                    keepdims=True)
        c = jnp.sum(jnp.where(onehot, 1.0, 0.0), axis=0, keepdims=True)
        acc_a = (a, c) if acc_a is None else (acc_a[0] + a, acc_a[1] + c)

    @pl.when(pl.program_id(0) == 0)
    def _():
        sums_ref[...] = jnp.zeros_like(sums_ref)
        cnt_ref[...] = jnp.zeros_like(cnt_ref)

    sums_ref[...] += acc_a[0] * (1.0 / _C)
    cnt_ref[...] += acc_a[1]


@jax.jit
def kernel(inputs, labels):
    labels2d = labels.astype(jnp.int32).reshape(_N, 1)
    sums, cnt = pl.pallas_call(
        _body,
        grid=(_G,),
        in_specs=[
            pl.BlockSpec((_B, 1), lambda i: (i, 0)),
            pl.BlockSpec((_B, 1), lambda i: (i + _G, 0)),
            pl.BlockSpec((_B, _C), lambda i: (i, 0)),
            pl.BlockSpec((_B, _C), lambda i: (i + _G, 0)),
        ],
        out_specs=[
            pl.BlockSpec((1, _C), lambda i: (0, 0)),
            pl.BlockSpec((1, _C), lambda i: (0, 0)),
        ],
        out_shape=[
            jax.ShapeDtypeStruct((1, _C), jnp.float32),
            jax.ShapeDtypeStruct((1, _C), jnp.float32),
        ],
    )(labels2d, labels2d, inputs, inputs)
    return (sums.reshape(_C), cnt.reshape(_C))
